# baseline (device time: 64307 ns/iter reference)
import os

import jax
import jax.numpy as jnp
from jax import lax
from jax.experimental import pallas as pl
from jax.experimental.pallas import tpu as pltpu

_NO_COMM = bool(int(os.environ.get("KERNEL_NO_COMM", "0")))

N_DEV = 4
B, SQ, SKV_G, HQ_G, DH = 2, 256, 1024, 16, 64
H_LOC = HQ_G // N_DEV
SKV_LOC = SKV_G // N_DEV
D_MODEL = 512
D_HEADS_LOC = H_LOC * DH
QB = SQ // 64
SQ_Q = SQ // N_DEV


def kernel(x, Wq, K_ext, V_ext, Wo):
    def body(x_ref, wq_ref, k_ref, v_ref, wo_ref, out_ref,
             kb16, vb16, k_rx, v_rx, part_buf, rs_rx, ag_rx,
             loc_sem,
             k_send, v_send, rs_send, ag_send,
             k_recv, v_recv, rs_recv, ag_recv):
        me = lax.axis_index("i")

        if not _NO_COMM:
            bar = pltpu.get_barrier_semaphore()
            for p in range(1, N_DEV):
                pl.semaphore_signal(
                    bar, inc=1,
                    device_id=((me + p) % N_DEV,),
                    device_id_type=pl.DeviceIdType.MESH,
                )
            pl.semaphore_wait(bar, N_DEV - 1)

        kb16[...] = k_ref[...].astype(jnp.bfloat16)
        vb16[...] = v_ref[...].astype(jnp.bfloat16)

        loc_cps = []
        for i, (st, rx) in enumerate(((kb16, k_rx), (vb16, v_rx))):
            for j in ([me] if not _NO_COMM else range(N_DEV)):
                cp = pltpu.make_async_copy(
                    st.at[:, :, pl.ds(H_LOC * me, H_LOC), :],
                    rx.at[j],
                    loc_sem.at[i],
                )
                cp.start()
                cp.wait()
                loc_cps.append(cp)

        sends = []
        if not _NO_COMM:
            for p in range(1, N_DEV):
                dst = (me + p) % N_DEV
                for (src_ref, rx, ssem, rsem) in (
                    (kb16, k_rx, k_send, k_recv),
                    (vb16, v_rx, v_send, v_recv),
                ):
                    rdma = pltpu.make_async_remote_copy(
                        src_ref=src_ref.at[:, :, pl.ds(H_LOC * dst, H_LOC), :],
                        dst_ref=rx.at[me],
                        send_sem=ssem.at[p],
                        recv_sem=rsem.at[me],
                        device_id=(dst,),
                        device_id_type=pl.DeviceIdType.MESH,
                    )
                    rdma.start()
                    sends.append(rdma)

        q = jnp.dot(
            x_ref[...].reshape(B * SQ, D_MODEL), wq_ref[...],
            preferred_element_type=jnp.float32,
        )
        qt = jnp.transpose(
            q.astype(jnp.bfloat16).reshape(B, SQ, H_LOC, DH), (0, 2, 1, 3))

        def slot_t(rx, s):
            blk = rx[pl.ds(s, 1)].reshape(B, SKV_LOC, H_LOC, DH)
            return jnp.transpose(blk, (0, 2, 1, 3))

        k_parts = [slot_t(k_rx, me)]
        v_parts = [slot_t(v_rx, me)]

        for p in ((1, 3, 2) if not _NO_COMM else (1, 2, 3)):
            src = (me + p) % N_DEV
            for (rx, ssem, rsem, parts) in (
                (k_rx, k_send, k_recv, k_parts),
                (v_rx, v_send, v_recv, v_parts),
            ):
                if not _NO_COMM:
                    rcv = pltpu.make_async_remote_copy(
                        src_ref=rx.at[src],
                        dst_ref=rx.at[src],
                        send_sem=ssem.at[0],
                        recv_sem=rsem.at[src],
                        device_id=(src,),
                        device_id_type=pl.DeviceIdType.MESH,
                    )
                    rcv.wait_recv()
                parts.append(slot_t(rx, src))

        kt = jnp.stack(k_parts, axis=2)
        vt = jnp.stack(v_parts, axis=2)

        for qb in range(QB):
            ctx_bs = []
            for b in range(B):
                q_blk = qt[b, :, 64 * qb:64 * (qb + 1), :]
                k_blk = kt[b, :, :, 64 * qb:64 * (qb + 1), :].reshape(
                    H_LOC, N_DEV * 64, DH)
                v_blk = vt[b, :, :, 64 * qb:64 * (qb + 1), :].reshape(
                    H_LOC, N_DEV * 64, DH)
                s = lax.dot_general(
                    q_blk, k_blk, (((2,), (2,)), ((0,), (0,))),
                    preferred_element_type=jnp.float32,
                ) * 0.125
                m = jnp.max(s, axis=-1, keepdims=True)
                w = jnp.exp(s - m)
                w = (w / jnp.sum(w, axis=-1, keepdims=True)).astype(
                    jnp.bfloat16)
                ctx = lax.dot_general(
                    w, v_blk, (((2,), (1,)), ((0,), (0,))),
                    preferred_element_type=jnp.float32,
                )
                ctx_bs.append(
                    jnp.transpose(ctx, (1, 0, 2)).reshape(64, D_HEADS_LOC))
            quarter = jnp.dot(
                jnp.concatenate(ctx_bs, axis=0), wo_ref[...],
                preferred_element_type=jnp.float32,
            ).reshape(B, SQ_Q, D_MODEL)
            part_buf[:, 64 * qb:64 * (qb + 1), :] = quarter
            if not _NO_COMM:
                @pl.when(qb != me)
                def _():
                    rdma = pltpu.make_async_remote_copy(
                        src_ref=part_buf.at[:, pl.ds(64 * qb, SQ_Q), :],
                        dst_ref=rs_rx.at[me],
                        send_sem=rs_send.at[qb],
                        recv_sem=rs_recv.at[me],
                        device_id=(qb,),
                        device_id_type=pl.DeviceIdType.MESH,
                    )
                    rdma.start()

        rs_rx[pl.ds(me, 1)] = part_buf[:, pl.ds(SQ_Q * me, SQ_Q), :][None]
        if _NO_COMM:
            for j in range(1, N_DEV):
                rs_rx[pl.ds((me + j) % N_DEV, 1)] = (
                    part_buf[:, pl.ds(SQ_Q * me, SQ_Q), :][None])
        else:
            for p in range(1, N_DEV):
                src = (me + p) % N_DEV
                rcv = pltpu.make_async_remote_copy(
                    src_ref=rs_rx.at[src],
                    dst_ref=rs_rx.at[src],
                    send_sem=rs_send.at[0],
                    recv_sem=rs_recv.at[src],
                    device_id=(src,),
                    device_id_type=pl.DeviceIdType.MESH,
                )
                rcv.wait_recv()

        r = rs_rx[...]
        red = r[0] + r[1] + r[2] + r[3]

        ag_rx[pl.ds(me, 1)] = red[None]
        if _NO_COMM:
            for j in range(1, N_DEV):
                ag_rx[pl.ds((me + j) % N_DEV, 1)] = red[None]
        else:
            for p in range(1, N_DEV):
                dst = (me + p) % N_DEV
                rdma = pltpu.make_async_remote_copy(
                    src_ref=ag_rx.at[me],
                    dst_ref=ag_rx.at[me],
                    send_sem=ag_send.at[p],
                    recv_sem=ag_recv.at[me],
                    device_id=(dst,),
                    device_id_type=pl.DeviceIdType.MESH,
                )
                rdma.start()
                sends.append(rdma)

            for p in range(1, N_DEV):
                src = (me + p) % N_DEV
                rcv = pltpu.make_async_remote_copy(
                    src_ref=ag_rx.at[src],
                    dst_ref=ag_rx.at[src],
                    send_sem=ag_send.at[0],
                    recv_sem=ag_recv.at[src],
                    device_id=(src,),
                    device_id_type=pl.DeviceIdType.MESH,
                )
                rcv.wait_recv()

        for j in range(N_DEV):
            out_ref[:, 64 * j:64 * (j + 1), :] = ag_rx[j]

        for rdma in sends:
            rdma.wait_send()
        if not _NO_COMM:
            for qb in range(QB):
                @pl.when(qb != me)
                def _():
                    rdma = pltpu.make_async_remote_copy(
                        src_ref=part_buf.at[:, pl.ds(64 * qb, SQ_Q), :],
                        dst_ref=rs_rx.at[me],
                        send_sem=rs_send.at[qb],
                        recv_sem=rs_recv.at[me],
                        device_id=(qb,),
                        device_id_type=pl.DeviceIdType.MESH,
                    )
                    rdma.wait_send()

    return pl.pallas_call(
        body,
        out_shape=jax.ShapeDtypeStruct((B, SQ, D_MODEL), jnp.float32),
        in_specs=[pl.BlockSpec(memory_space=pltpu.VMEM)] * 5,
        out_specs=pl.BlockSpec(memory_space=pltpu.VMEM),
        scratch_shapes=[
            pltpu.VMEM((B, SKV_LOC, HQ_G, DH), jnp.bfloat16),
            pltpu.VMEM((B, SKV_LOC, HQ_G, DH), jnp.bfloat16),
            pltpu.VMEM((N_DEV, B, SKV_LOC, H_LOC, DH), jnp.bfloat16),
            pltpu.VMEM((N_DEV, B, SKV_LOC, H_LOC, DH), jnp.bfloat16),
            pltpu.VMEM((B, SQ, D_MODEL), jnp.float32),
            pltpu.VMEM((N_DEV, B, SQ_Q, D_MODEL), jnp.float32),
            pltpu.VMEM((N_DEV, B, SQ_Q, D_MODEL), jnp.float32),
            pltpu.SemaphoreType.DMA((2,)),
            pltpu.SemaphoreType.DMA((N_DEV,)),
            pltpu.SemaphoreType.DMA((N_DEV,)),
            pltpu.SemaphoreType.DMA((N_DEV,)),
            pltpu.SemaphoreType.DMA((N_DEV,)),
            pltpu.SemaphoreType.DMA((N_DEV,)),
            pltpu.SemaphoreType.DMA((N_DEV,)),
            pltpu.SemaphoreType.DMA((N_DEV,)),
            pltpu.SemaphoreType.DMA((N_DEV,)),
        ],
        compiler_params=pltpu.CompilerParams(
            collective_id=None if _NO_COMM else 0,
            vmem_limit_bytes=100 * 1024 * 1024,
        ),
    )(x, Wq, K_ext, V_ext, Wo)


# device time: 58698 ns/iter; 1.0956x vs baseline; 1.0956x over previous
import os

import jax
import jax.numpy as jnp
from jax import lax
from jax.experimental import pallas as pl
from jax.experimental.pallas import tpu as pltpu

_NO_COMM = bool(int(os.environ.get("KERNEL_NO_COMM", "0")))

N_DEV = 4
B, SQ, SKV_G, HQ_G, DH = 2, 256, 1024, 16, 64
H_LOC = HQ_G // N_DEV
SKV_LOC = SKV_G // N_DEV
D_MODEL = 512
D_HEADS_LOC = H_LOC * DH
QB = SQ // 64
SQ_Q = SQ // N_DEV


def kernel(x, Wq, K_ext, V_ext, Wo):
    def body(x_ref, wq_ref, k_ref, v_ref, wo_ref, out_ref,
             kb16, vb16, k_rx, v_rx, part_buf, rs_rx, ag_rx,
             loc_sem,
             k_send, v_send, rs_send, ag_send,
             k_recv, v_recv, rs_recv, ag_recv):
        me = lax.axis_index("i")

        if not _NO_COMM:
            bar = pltpu.get_barrier_semaphore()
            for p in range(1, N_DEV):
                pl.semaphore_signal(
                    bar, inc=1,
                    device_id=((me + p) % N_DEV,),
                    device_id_type=pl.DeviceIdType.MESH,
                )
            pl.semaphore_wait(bar, N_DEV - 1)

        kb16[...] = k_ref[...].astype(jnp.bfloat16)
        vb16[...] = v_ref[...].astype(jnp.bfloat16)

        loc_cps = []
        for i, (st, rx) in enumerate(((kb16, k_rx), (vb16, v_rx))):
            for j in ([me] if not _NO_COMM else range(N_DEV)):
                cp = pltpu.make_async_copy(
                    st.at[:, :, pl.ds(H_LOC * me, H_LOC), :],
                    rx.at[j],
                    loc_sem.at[i],
                )
                cp.start()
                cp.wait()
                loc_cps.append(cp)

        sends = []
        if not _NO_COMM:
            for p in range(1, N_DEV):
                dst = (me + p) % N_DEV
                for (src_ref, rx, ssem, rsem) in (
                    (kb16, k_rx, k_send, k_recv),
                    (vb16, v_rx, v_send, v_recv),
                ):
                    rdma = pltpu.make_async_remote_copy(
                        src_ref=src_ref.at[:, :, pl.ds(H_LOC * dst, H_LOC), :],
                        dst_ref=rx.at[me],
                        send_sem=ssem.at[p],
                        recv_sem=rsem.at[me],
                        device_id=(dst,),
                        device_id_type=pl.DeviceIdType.MESH,
                    )
                    rdma.start()
                    sends.append(rdma)

        q = jnp.dot(
            x_ref[...].reshape(B * SQ, D_MODEL).astype(jnp.bfloat16),
            wq_ref[...].astype(jnp.bfloat16),
            preferred_element_type=jnp.float32,
        )
        qt = jnp.transpose(
            q.astype(jnp.bfloat16).reshape(B, SQ, H_LOC, DH), (0, 2, 1, 3))
        wo16 = wo_ref[...].astype(jnp.bfloat16)

        def slot_t(rx, s):
            blk = rx[pl.ds(s, 1)].reshape(B, SKV_LOC, H_LOC, DH)
            return jnp.transpose(blk, (0, 2, 1, 3))

        k_parts = [slot_t(k_rx, me)]
        v_parts = [slot_t(v_rx, me)]

        for p in ((1, 3, 2) if not _NO_COMM else (1, 2, 3)):
            src = (me + p) % N_DEV
            for (rx, ssem, rsem, parts) in (
                (k_rx, k_send, k_recv, k_parts),
                (v_rx, v_send, v_recv, v_parts),
            ):
                if not _NO_COMM:
                    rcv = pltpu.make_async_remote_copy(
                        src_ref=rx.at[src],
                        dst_ref=rx.at[src],
                        send_sem=ssem.at[0],
                        recv_sem=rsem.at[src],
                        device_id=(src,),
                        device_id_type=pl.DeviceIdType.MESH,
                    )
                    rcv.wait_recv()
                parts.append(slot_t(rx, src))

        kt = jnp.stack(k_parts, axis=2)
        vt = jnp.stack(v_parts, axis=2)

        for qb in range(QB):
            ctx_bs = []
            for b in range(B):
                q_blk = qt[b, :, 64 * qb:64 * (qb + 1), :]
                k_blk = kt[b, :, :, 64 * qb:64 * (qb + 1), :].reshape(
                    H_LOC, N_DEV * 64, DH)
                v_blk = vt[b, :, :, 64 * qb:64 * (qb + 1), :].reshape(
                    H_LOC, N_DEV * 64, DH)
                s = lax.dot_general(
                    q_blk, k_blk, (((2,), (2,)), ((0,), (0,))),
                    preferred_element_type=jnp.float32,
                ) * 0.125
                m = jnp.max(s, axis=-1, keepdims=True)
                w = jnp.exp(s - m)
                w = (w / jnp.sum(w, axis=-1, keepdims=True)).astype(
                    jnp.bfloat16)
                ctx = lax.dot_general(
                    w, v_blk, (((2,), (1,)), ((0,), (0,))),
                    preferred_element_type=jnp.float32,
                )
                ctx_bs.append(
                    jnp.transpose(ctx, (1, 0, 2)).reshape(64, D_HEADS_LOC))
            quarter = jnp.dot(
                jnp.concatenate(ctx_bs, axis=0).astype(jnp.bfloat16), wo16,
                preferred_element_type=jnp.float32,
            ).reshape(B, SQ_Q, D_MODEL)
            part_buf[:, 64 * qb:64 * (qb + 1), :] = quarter.astype(
                jnp.bfloat16)
            if not _NO_COMM:
                @pl.when(qb != me)
                def _():
                    rdma = pltpu.make_async_remote_copy(
                        src_ref=part_buf.at[:, pl.ds(64 * qb, SQ_Q), :],
                        dst_ref=rs_rx.at[me],
                        send_sem=rs_send.at[qb],
                        recv_sem=rs_recv.at[me],
                        device_id=(qb,),
                        device_id_type=pl.DeviceIdType.MESH,
                    )
                    rdma.start()

        rs_rx[pl.ds(me, 1)] = part_buf[:, pl.ds(SQ_Q * me, SQ_Q), :][None]
        if _NO_COMM:
            for j in range(1, N_DEV):
                rs_rx[pl.ds((me + j) % N_DEV, 1)] = (
                    part_buf[:, pl.ds(SQ_Q * me, SQ_Q), :][None])
        else:
            for p in range(1, N_DEV):
                src = (me + p) % N_DEV
                rcv = pltpu.make_async_remote_copy(
                    src_ref=rs_rx.at[src],
                    dst_ref=rs_rx.at[src],
                    send_sem=rs_send.at[0],
                    recv_sem=rs_recv.at[src],
                    device_id=(src,),
                    device_id_type=pl.DeviceIdType.MESH,
                )
                rcv.wait_recv()

        r = rs_rx[...].astype(jnp.float32)
        red = (r[0] + r[1] + r[2] + r[3]).astype(jnp.bfloat16)

        ag_rx[pl.ds(me, 1)] = red[None]
        if _NO_COMM:
            for j in range(1, N_DEV):
                ag_rx[pl.ds((me + j) % N_DEV, 1)] = red[None]
        else:
            for p in range(1, N_DEV):
                dst = (me + p) % N_DEV
                rdma = pltpu.make_async_remote_copy(
                    src_ref=ag_rx.at[me],
                    dst_ref=ag_rx.at[me],
                    send_sem=ag_send.at[p],
                    recv_sem=ag_recv.at[me],
                    device_id=(dst,),
                    device_id_type=pl.DeviceIdType.MESH,
                )
                rdma.start()
                sends.append(rdma)

            for p in range(1, N_DEV):
                src = (me + p) % N_DEV
                rcv = pltpu.make_async_remote_copy(
                    src_ref=ag_rx.at[src],
                    dst_ref=ag_rx.at[src],
                    send_sem=ag_send.at[0],
                    recv_sem=ag_recv.at[src],
                    device_id=(src,),
                    device_id_type=pl.DeviceIdType.MESH,
                )
                rcv.wait_recv()

        for j in range(N_DEV):
            out_ref[:, 64 * j:64 * (j + 1), :] = ag_rx[j].astype(jnp.float32)

        for rdma in sends:
            rdma.wait_send()
        if not _NO_COMM:
            for qb in range(QB):
                @pl.when(qb != me)
                def _():
                    rdma = pltpu.make_async_remote_copy(
                        src_ref=part_buf.at[:, pl.ds(64 * qb, SQ_Q), :],
                        dst_ref=rs_rx.at[me],
                        send_sem=rs_send.at[qb],
                        recv_sem=rs_recv.at[me],
                        device_id=(qb,),
                        device_id_type=pl.DeviceIdType.MESH,
                    )
                    rdma.wait_send()

    return pl.pallas_call(
        body,
        out_shape=jax.ShapeDtypeStruct((B, SQ, D_MODEL), jnp.float32),
        in_specs=[pl.BlockSpec(memory_space=pltpu.VMEM)] * 5,
        out_specs=pl.BlockSpec(memory_space=pltpu.VMEM),
        scratch_shapes=[
            pltpu.VMEM((B, SKV_LOC, HQ_G, DH), jnp.bfloat16),
            pltpu.VMEM((B, SKV_LOC, HQ_G, DH), jnp.bfloat16),
            pltpu.VMEM((N_DEV, B, SKV_LOC, H_LOC, DH), jnp.bfloat16),
            pltpu.VMEM((N_DEV, B, SKV_LOC, H_LOC, DH), jnp.bfloat16),
            pltpu.VMEM((B, SQ, D_MODEL), jnp.bfloat16),
            pltpu.VMEM((N_DEV, B, SQ_Q, D_MODEL), jnp.bfloat16),
            pltpu.VMEM((N_DEV, B, SQ_Q, D_MODEL), jnp.bfloat16),
            pltpu.SemaphoreType.DMA((2,)),
            pltpu.SemaphoreType.DMA((N_DEV,)),
            pltpu.SemaphoreType.DMA((N_DEV,)),
            pltpu.SemaphoreType.DMA((N_DEV,)),
            pltpu.SemaphoreType.DMA((N_DEV,)),
            pltpu.SemaphoreType.DMA((N_DEV,)),
            pltpu.SemaphoreType.DMA((N_DEV,)),
            pltpu.SemaphoreType.DMA((N_DEV,)),
            pltpu.SemaphoreType.DMA((N_DEV,)),
        ],
        compiler_params=pltpu.CompilerParams(
            collective_id=None if _NO_COMM else 0,
            vmem_limit_bytes=100 * 1024 * 1024,
        ),
    )(x, Wq, K_ext, V_ext, Wo)


# device time: 55989 ns/iter; 1.1486x vs baseline; 1.0484x over previous
import os

import jax
import jax.numpy as jnp
from jax import lax
from jax.experimental import pallas as pl
from jax.experimental.pallas import tpu as pltpu

_NO_COMM = bool(int(os.environ.get("KERNEL_NO_COMM", "0")))

N_DEV = 4
B, SQ, SKV_G, HQ_G, DH = 2, 256, 1024, 16, 64
H_LOC = HQ_G // N_DEV
SKV_LOC = SKV_G // N_DEV
D_MODEL = 512
D_HEADS_LOC = H_LOC * DH
QB = SQ // 64
SQ_Q = SQ // N_DEV


def kernel(x, Wq, K_ext, V_ext, Wo):
    def body(x_ref, wq_ref, k_ref, v_ref, wo_ref, out_ref,
             kb16, vb16, k_rx, v_rx, part_buf, rs_rx, ag_rx,
             loc_sem,
             k_send, v_send, rs_send, ag_send,
             k_recv, v_recv, rs_recv, ag_recv):
        me = lax.axis_index("i")

        if not _NO_COMM:
            bar = pltpu.get_barrier_semaphore()
            for p in range(1, N_DEV):
                pl.semaphore_signal(
                    bar, inc=1,
                    device_id=((me + p) % N_DEV,),
                    device_id_type=pl.DeviceIdType.MESH,
                )
            pl.semaphore_wait(bar, N_DEV - 1)

        kb16[...] = k_ref[...].astype(jnp.bfloat16)
        vb16[...] = v_ref[...].astype(jnp.bfloat16)

        loc_cps = []
        for i, (st, rx) in enumerate(((kb16, k_rx), (vb16, v_rx))):
            for j in ([me] if not _NO_COMM else range(N_DEV)):
                cp = pltpu.make_async_copy(
                    st.at[:, :, pl.ds(H_LOC * me, H_LOC), :],
                    rx.at[j],
                    loc_sem.at[i],
                )
                cp.start()
                cp.wait()
                loc_cps.append(cp)

        sends = []
        if not _NO_COMM:
            for (src_ref, rx, ssem, rsem) in (
                (kb16, k_rx, k_send, k_recv),
                (vb16, v_rx, v_send, v_recv),
            ):
                for p in range(1, N_DEV):
                    dst = (me + p) % N_DEV
                    rdma = pltpu.make_async_remote_copy(
                        src_ref=src_ref.at[:, :, pl.ds(H_LOC * dst, H_LOC), :],
                        dst_ref=rx.at[me],
                        send_sem=ssem.at[p],
                        recv_sem=rsem.at[me],
                        device_id=(dst,),
                        device_id_type=pl.DeviceIdType.MESH,
                    )
                    rdma.start()
                    sends.append(rdma)

        q = jnp.dot(
            x_ref[...].reshape(B * SQ, D_MODEL).astype(jnp.bfloat16),
            wq_ref[...].astype(jnp.bfloat16),
            preferred_element_type=jnp.float32,
        )
        qt = jnp.transpose(
            q.astype(jnp.bfloat16).reshape(B, SQ, H_LOC, DH), (0, 2, 1, 3))
        wo16 = wo_ref[...].astype(jnp.bfloat16)

        def slot_t(rx, s):
            blk = rx[pl.ds(s, 1)].reshape(B, SKV_LOC, H_LOC, DH)
            return jnp.transpose(blk, (0, 2, 1, 3))

        def wait_slot(rx, ssem, rsem, src):
            rcv = pltpu.make_async_remote_copy(
                src_ref=rx.at[src],
                dst_ref=rx.at[src],
                send_sem=ssem.at[0],
                recv_sem=rsem.at[src],
                device_id=(src,),
                device_id_type=pl.DeviceIdType.MESH,
            )
            rcv.wait_recv()

        wait_order = (1, 3, 2)
        k_parts = [slot_t(k_rx, me)]
        for p in wait_order:
            src = (me + p) % N_DEV
            if not _NO_COMM:
                wait_slot(k_rx, k_send, k_recv, src)
            k_parts.append(slot_t(k_rx, src))
        kt = jnp.stack(k_parts, axis=2)

        ws = {}
        for qb in range(QB):
            for b in range(B):
                q_blk = qt[b, :, 64 * qb:64 * (qb + 1), :]
                k_blk = kt[b, :, :, 64 * qb:64 * (qb + 1), :].reshape(
                    H_LOC, N_DEV * 64, DH)
                s = lax.dot_general(
                    q_blk, k_blk, (((2,), (2,)), ((0,), (0,))),
                    preferred_element_type=jnp.float32,
                ) * 0.125
                m = jnp.max(s, axis=-1, keepdims=True)
                w = jnp.exp(s - m)
                ws[qb, b] = (w / jnp.sum(w, axis=-1, keepdims=True)).astype(
                    jnp.bfloat16)

        v_parts = [slot_t(v_rx, me)]
        for p in wait_order:
            src = (me + p) % N_DEV
            if not _NO_COMM:
                wait_slot(v_rx, v_send, v_recv, src)
            v_parts.append(slot_t(v_rx, src))
        vt = jnp.stack(v_parts, axis=2)

        for qb in range(QB):
            ctx_bs = []
            for b in range(B):
                v_blk = vt[b, :, :, 64 * qb:64 * (qb + 1), :].reshape(
                    H_LOC, N_DEV * 64, DH)
                ctx = lax.dot_general(
                    ws[qb, b], v_blk, (((2,), (1,)), ((0,), (0,))),
                    preferred_element_type=jnp.float32,
                )
                ctx_bs.append(
                    jnp.transpose(ctx, (1, 0, 2)).reshape(64, D_HEADS_LOC))
            quarter = jnp.dot(
                jnp.concatenate(ctx_bs, axis=0).astype(jnp.bfloat16), wo16,
                preferred_element_type=jnp.float32,
            ).reshape(B, SQ_Q, D_MODEL)
            part_buf[:, 64 * qb:64 * (qb + 1), :] = quarter.astype(
                jnp.bfloat16)
            if not _NO_COMM:
                @pl.when(qb != me)
                def _():
                    rdma = pltpu.make_async_remote_copy(
                        src_ref=part_buf.at[:, pl.ds(64 * qb, SQ_Q), :],
                        dst_ref=rs_rx.at[me],
                        send_sem=rs_send.at[qb],
                        recv_sem=rs_recv.at[me],
                        device_id=(qb,),
                        device_id_type=pl.DeviceIdType.MESH,
                    )
                    rdma.start()

        rs_rx[pl.ds(me, 1)] = part_buf[:, pl.ds(SQ_Q * me, SQ_Q), :][None]
        if _NO_COMM:
            for j in range(1, N_DEV):
                rs_rx[pl.ds((me + j) % N_DEV, 1)] = (
                    part_buf[:, pl.ds(SQ_Q * me, SQ_Q), :][None])
        else:
            for p in range(1, N_DEV):
                src = (me + p) % N_DEV
                rcv = pltpu.make_async_remote_copy(
                    src_ref=rs_rx.at[src],
                    dst_ref=rs_rx.at[src],
                    send_sem=rs_send.at[0],
                    recv_sem=rs_recv.at[src],
                    device_id=(src,),
                    device_id_type=pl.DeviceIdType.MESH,
                )
                rcv.wait_recv()

        r = rs_rx[...].astype(jnp.float32)
        red = (r[0] + r[1] + r[2] + r[3]).astype(jnp.bfloat16)

        ag_rx[pl.ds(me, 1)] = red[None]
        if _NO_COMM:
            for j in range(1, N_DEV):
                ag_rx[pl.ds((me + j) % N_DEV, 1)] = red[None]
        else:
            for p in range(1, N_DEV):
                dst = (me + p) % N_DEV
                rdma = pltpu.make_async_remote_copy(
                    src_ref=ag_rx.at[me],
                    dst_ref=ag_rx.at[me],
                    send_sem=ag_send.at[p],
                    recv_sem=ag_recv.at[me],
                    device_id=(dst,),
                    device_id_type=pl.DeviceIdType.MESH,
                )
                rdma.start()
                sends.append(rdma)

            for p in range(1, N_DEV):
                src = (me + p) % N_DEV
                rcv = pltpu.make_async_remote_copy(
                    src_ref=ag_rx.at[src],
                    dst_ref=ag_rx.at[src],
                    send_sem=ag_send.at[0],
                    recv_sem=ag_recv.at[src],
                    device_id=(src,),
                    device_id_type=pl.DeviceIdType.MESH,
                )
                rcv.wait_recv()

        for j in range(N_DEV):
            out_ref[:, 64 * j:64 * (j + 1), :] = ag_rx[j].astype(jnp.float32)

        for rdma in sends:
            rdma.wait_send()
        if not _NO_COMM:
            for qb in range(QB):
                @pl.when(qb != me)
                def _():
                    rdma = pltpu.make_async_remote_copy(
                        src_ref=part_buf.at[:, pl.ds(64 * qb, SQ_Q), :],
                        dst_ref=rs_rx.at[me],
                        send_sem=rs_send.at[qb],
                        recv_sem=rs_recv.at[me],
                        device_id=(qb,),
                        device_id_type=pl.DeviceIdType.MESH,
                    )
                    rdma.wait_send()

    return pl.pallas_call(
        body,
        out_shape=jax.ShapeDtypeStruct((B, SQ, D_MODEL), jnp.float32),
        in_specs=[pl.BlockSpec(memory_space=pltpu.VMEM)] * 5,
        out_specs=pl.BlockSpec(memory_space=pltpu.VMEM),
        scratch_shapes=[
            pltpu.VMEM((B, SKV_LOC, HQ_G, DH), jnp.bfloat16),
            pltpu.VMEM((B, SKV_LOC, HQ_G, DH), jnp.bfloat16),
            pltpu.VMEM((N_DEV, B, SKV_LOC, H_LOC, DH), jnp.bfloat16),
            pltpu.VMEM((N_DEV, B, SKV_LOC, H_LOC, DH), jnp.bfloat16),
            pltpu.VMEM((B, SQ, D_MODEL), jnp.bfloat16),
            pltpu.VMEM((N_DEV, B, SQ_Q, D_MODEL), jnp.bfloat16),
            pltpu.VMEM((N_DEV, B, SQ_Q, D_MODEL), jnp.bfloat16),
            pltpu.SemaphoreType.DMA((2,)),
            pltpu.SemaphoreType.DMA((N_DEV,)),
            pltpu.SemaphoreType.DMA((N_DEV,)),
            pltpu.SemaphoreType.DMA((N_DEV,)),
            pltpu.SemaphoreType.DMA((N_DEV,)),
            pltpu.SemaphoreType.DMA((N_DEV,)),
            pltpu.SemaphoreType.DMA((N_DEV,)),
            pltpu.SemaphoreType.DMA((N_DEV,)),
            pltpu.SemaphoreType.DMA((N_DEV,)),
        ],
        compiler_params=pltpu.CompilerParams(
            collective_id=None if _NO_COMM else 0,
            vmem_limit_bytes=100 * 1024 * 1024,
        ),
    )(x, Wq, K_ext, V_ext, Wo)


# device time: 54801 ns/iter; 1.1735x vs baseline; 1.0217x over previous
import os

import jax
import jax.numpy as jnp
from jax import lax
from jax.experimental import pallas as pl
from jax.experimental.pallas import tpu as pltpu

_NO_COMM = bool(int(os.environ.get("KERNEL_NO_COMM", "0")))

N_DEV = 4
B, SQ, SKV_G, HQ_G, DH = 2, 256, 1024, 16, 64
H_LOC = HQ_G // N_DEV
SKV_LOC = SKV_G // N_DEV
D_MODEL = 512
D_HEADS_LOC = H_LOC * DH
QB = SQ // 64
SQ_Q = SQ // N_DEV


def kernel(x, Wq, K_ext, V_ext, Wo):
    def body(x_ref, wq_ref, k_ref, v_ref, wo_ref, out_ref,
             kb16, vb16, k_rx, v_rx, part_buf, rs_rx, ag_rx,
             loc_sem,
             k_send, v_send, rs_send, ag_send,
             k_recv, v_recv, rs_recv, ag_recv):
        me = lax.axis_index("i")

        if not _NO_COMM:
            bar = pltpu.get_barrier_semaphore()
            for p in range(1, N_DEV):
                pl.semaphore_signal(
                    bar, inc=1,
                    device_id=((me + p) % N_DEV,),
                    device_id_type=pl.DeviceIdType.MESH,
                )
            pl.semaphore_wait(bar, N_DEV - 1)

        kb16[...] = k_ref[...].astype(jnp.bfloat16)
        vb16[...] = v_ref[...].astype(jnp.bfloat16)

        loc_cps = []
        for i, (st, rx) in enumerate(((kb16, k_rx), (vb16, v_rx))):
            for j in ([me] if not _NO_COMM else range(N_DEV)):
                cp = pltpu.make_async_copy(
                    st.at[:, :, pl.ds(H_LOC * me, H_LOC), :],
                    rx.at[j],
                    loc_sem.at[i],
                )
                cp.start()
                cp.wait()
                loc_cps.append(cp)

        sends = []
        if not _NO_COMM:
            for (src_ref, rx, ssem, rsem) in (
                (kb16, k_rx, k_send, k_recv),
                (vb16, v_rx, v_send, v_recv),
            ):
                for p in (1, 3, 2):
                    dst = (me + p) % N_DEV
                    rdma = pltpu.make_async_remote_copy(
                        src_ref=src_ref.at[:, :, pl.ds(H_LOC * dst, H_LOC), :],
                        dst_ref=rx.at[me],
                        send_sem=ssem.at[p],
                        recv_sem=rsem.at[me],
                        device_id=(dst,),
                        device_id_type=pl.DeviceIdType.MESH,
                    )
                    rdma.start()
                    sends.append(rdma)

        q = jnp.dot(
            x_ref[...].reshape(B * SQ, D_MODEL).astype(jnp.bfloat16),
            wq_ref[...].astype(jnp.bfloat16),
            preferred_element_type=jnp.float32,
        )
        qt = jnp.transpose(
            q.astype(jnp.bfloat16).reshape(B, SQ, H_LOC, DH), (0, 2, 1, 3))
        wo16 = wo_ref[...].astype(jnp.bfloat16)

        def slot_t(rx, s):
            blk = rx[pl.ds(s, 1)].reshape(B, SKV_LOC, H_LOC, DH)
            return jnp.transpose(blk, (0, 2, 1, 3))

        def wait_slot(rx, ssem, rsem, src):
            rcv = pltpu.make_async_remote_copy(
                src_ref=rx.at[src],
                dst_ref=rx.at[src],
                send_sem=ssem.at[0],
                recv_sem=rsem.at[src],
                device_id=(src,),
                device_id_type=pl.DeviceIdType.MESH,
            )
            rcv.wait_recv()

        wait_order = (1, 3, 2)
        k_parts = [slot_t(k_rx, me)]
        for p in wait_order:
            src = (me + p) % N_DEV
            if not _NO_COMM:
                wait_slot(k_rx, k_send, k_recv, src)
            k_parts.append(slot_t(k_rx, src))
        kt = jnp.stack(k_parts, axis=2)

        ws = {}
        for qb in range(QB):
            for b in range(B):
                q_blk = qt[b, :, 64 * qb:64 * (qb + 1), :]
                k_blk = kt[b, :, :, 64 * qb:64 * (qb + 1), :].reshape(
                    H_LOC, N_DEV * 64, DH)
                s = lax.dot_general(
                    q_blk, k_blk, (((2,), (2,)), ((0,), (0,))),
                    preferred_element_type=jnp.float32,
                ) * 0.125
                e = jnp.exp(s)
                ws[qb, b] = (e.astype(jnp.bfloat16),
                             jnp.sum(e, axis=-1, keepdims=True))

        v_parts = [slot_t(v_rx, me)]
        for p in wait_order:
            src = (me + p) % N_DEV
            if not _NO_COMM:
                wait_slot(v_rx, v_send, v_recv, src)
            v_parts.append(slot_t(v_rx, src))
        vt = jnp.stack(v_parts, axis=2)

        for qb in range(QB):
            ctx_bs = []
            for b in range(B):
                v_blk = vt[b, :, :, 64 * qb:64 * (qb + 1), :].reshape(
                    H_LOC, N_DEV * 64, DH)
                e16, denom = ws[qb, b]
                ctx = lax.dot_general(
                    e16, v_blk, (((2,), (1,)), ((0,), (0,))),
                    preferred_element_type=jnp.float32,
                ) / denom
                ctx_bs.append(
                    jnp.transpose(ctx, (1, 0, 2)).reshape(64, D_HEADS_LOC))
            quarter = jnp.dot(
                jnp.concatenate(ctx_bs, axis=0).astype(jnp.bfloat16), wo16,
                preferred_element_type=jnp.float32,
            ).reshape(B, SQ_Q, D_MODEL)
            part_buf[:, 64 * qb:64 * (qb + 1), :] = quarter.astype(
                jnp.bfloat16)
            if not _NO_COMM:
                @pl.when(qb != me)
                def _():
                    rdma = pltpu.make_async_remote_copy(
                        src_ref=part_buf.at[:, pl.ds(64 * qb, SQ_Q), :],
                        dst_ref=rs_rx.at[me],
                        send_sem=rs_send.at[qb],
                        recv_sem=rs_recv.at[me],
                        device_id=(qb,),
                        device_id_type=pl.DeviceIdType.MESH,
                    )
                    rdma.start()

        rs_rx[pl.ds(me, 1)] = part_buf[:, pl.ds(SQ_Q * me, SQ_Q), :][None]
        if _NO_COMM:
            for j in range(1, N_DEV):
                rs_rx[pl.ds((me + j) % N_DEV, 1)] = (
                    part_buf[:, pl.ds(SQ_Q * me, SQ_Q), :][None])
        else:
            for p in range(1, N_DEV):
                src = (me + p) % N_DEV
                rcv = pltpu.make_async_remote_copy(
                    src_ref=rs_rx.at[src],
                    dst_ref=rs_rx.at[src],
                    send_sem=rs_send.at[0],
                    recv_sem=rs_recv.at[src],
                    device_id=(src,),
                    device_id_type=pl.DeviceIdType.MESH,
                )
                rcv.wait_recv()

        r = rs_rx[...].astype(jnp.float32)
        red = (r[0] + r[1] + r[2] + r[3]).astype(jnp.bfloat16)

        ag_rx[pl.ds(me, 1)] = red[None]
        if _NO_COMM:
            for j in range(1, N_DEV):
                ag_rx[pl.ds((me + j) % N_DEV, 1)] = red[None]
        else:
            for p in range(1, N_DEV):
                dst = (me + p) % N_DEV
                rdma = pltpu.make_async_remote_copy(
                    src_ref=ag_rx.at[me],
                    dst_ref=ag_rx.at[me],
                    send_sem=ag_send.at[p],
                    recv_sem=ag_recv.at[me],
                    device_id=(dst,),
                    device_id_type=pl.DeviceIdType.MESH,
                )
                rdma.start()
                sends.append(rdma)

            for p in range(1, N_DEV):
                src = (me + p) % N_DEV
                rcv = pltpu.make_async_remote_copy(
                    src_ref=ag_rx.at[src],
                    dst_ref=ag_rx.at[src],
                    send_sem=ag_send.at[0],
                    recv_sem=ag_recv.at[src],
                    device_id=(src,),
                    device_id_type=pl.DeviceIdType.MESH,
                )
                rcv.wait_recv()

        for j in range(N_DEV):
            out_ref[:, 64 * j:64 * (j + 1), :] = ag_rx[j].astype(jnp.float32)

        for rdma in sends:
            rdma.wait_send()
        if not _NO_COMM:
            for qb in range(QB):
                @pl.when(qb != me)
                def _():
                    rdma = pltpu.make_async_remote_copy(
                        src_ref=part_buf.at[:, pl.ds(64 * qb, SQ_Q), :],
                        dst_ref=rs_rx.at[me],
                        send_sem=rs_send.at[qb],
                        recv_sem=rs_recv.at[me],
                        device_id=(qb,),
                        device_id_type=pl.DeviceIdType.MESH,
                    )
                    rdma.wait_send()

    return pl.pallas_call(
        body,
        out_shape=jax.ShapeDtypeStruct((B, SQ, D_MODEL), jnp.float32),
        in_specs=[pl.BlockSpec(memory_space=pltpu.VMEM)] * 5,
        out_specs=pl.BlockSpec(memory_space=pltpu.VMEM),
        scratch_shapes=[
            pltpu.VMEM((B, SKV_LOC, HQ_G, DH), jnp.bfloat16),
            pltpu.VMEM((B, SKV_LOC, HQ_G, DH), jnp.bfloat16),
            pltpu.VMEM((N_DEV, B, SKV_LOC, H_LOC, DH), jnp.bfloat16),
            pltpu.VMEM((N_DEV, B, SKV_LOC, H_LOC, DH), jnp.bfloat16),
            pltpu.VMEM((B, SQ, D_MODEL), jnp.bfloat16),
            pltpu.VMEM((N_DEV, B, SQ_Q, D_MODEL), jnp.bfloat16),
            pltpu.VMEM((N_DEV, B, SQ_Q, D_MODEL), jnp.bfloat16),
            pltpu.SemaphoreType.DMA((2,)),
            pltpu.SemaphoreType.DMA((N_DEV,)),
            pltpu.SemaphoreType.DMA((N_DEV,)),
            pltpu.SemaphoreType.DMA((N_DEV,)),
            pltpu.SemaphoreType.DMA((N_DEV,)),
            pltpu.SemaphoreType.DMA((N_DEV,)),
            pltpu.SemaphoreType.DMA((N_DEV,)),
            pltpu.SemaphoreType.DMA((N_DEV,)),
            pltpu.SemaphoreType.DMA((N_DEV,)),
        ],
        compiler_params=pltpu.CompilerParams(
            collective_id=None if _NO_COMM else 0,
            vmem_limit_bytes=100 * 1024 * 1024,
        ),
    )(x, Wq, K_ext, V_ext, Wo)
